# Initial kernel scaffold; baseline (speedup 1.0000x reference)
import jax
import jax.numpy as jnp
from jax.experimental import pallas as pl


def kernel(z, edge_index):
    raise NotImplementedError("write your pallas kernel here")


# speedup vs baseline: 2.7093x; 2.7093x over previous
import functools

import jax
import jax.numpy as jnp
from jax import lax
from jax.experimental import pallas as pl
from jax.experimental.pallas import tpu as pltpu
from jax.experimental.pallas import tpu_sc as plsc

N = 10000
D = 256
E = 160000
L = 16
NC = 2
NS = 16
NW = NC * NS
C = 128
NCHUNKS = E // C
JB = D // L


def _normalize_body(z_ref, out_ref):
    z = z_ref[...]
    ss = jnp.sum(z * z, axis=1, keepdims=True)
    inv = 1.0 / jnp.maximum(jnp.sqrt(ss), 1e-8)
    out_ref[...] = z * inv


def _normalize(z):
    return pl.pallas_call(
        _normalize_body,
        out_shape=jax.ShapeDtypeStruct((N, D), jnp.float32),
    )(z)


def _sc_body(zn_hbm, src_hbm, dst_hbm, out_hbm,
             idxa_v, idxb_v, a_v, b_v, p_v, out_v, sem):
    wid = lax.axis_index("s") * NC + lax.axis_index("c")
    nchunks_w = (NCHUNKS - wid + NW - 1) // NW

    def chunk_body(i, _):
        cb = (wid + i * NW) * C
        pltpu.sync_copy(src_hbm.at[pl.ds(cb, C)], idxa_v)
        pltpu.sync_copy(dst_hbm.at[pl.ds(cb, C)], idxb_v)
        cpa = pltpu.async_copy(zn_hbm.at[idxa_v], a_v, sem)
        cpb = pltpu.async_copy(zn_hbm.at[idxb_v], b_v, sem)
        cpa.wait()
        cpb.wait()

        def edge_body(e, _):
            acc = a_v[e, pl.ds(0, L)] * b_v[e, pl.ds(0, L)]
            for j in range(1, JB):
                acc = acc + a_v[e, pl.ds(j * L, L)] * b_v[e, pl.ds(j * L, L)]
            p_v[pl.ds(e * L, L)] = acc
            return 0

        lax.fori_loop(0, C, edge_body, 0)

        lanes = lax.iota(jnp.int32, L)
        for g in range(C // L):
            base_idx = (lanes + g * L) * L
            acc = plsc.load_gather(p_v, [base_idx])
            for d in range(1, L):
                acc = acc + plsc.load_gather(p_v, [base_idx + d])
            sig = 1.0 / (1.0 + jnp.exp(-acc))
            out_v[pl.ds(g * L, L)] = sig

        pltpu.sync_copy(out_v, out_hbm.at[pl.ds(cb, C)])
        return 0

    lax.fori_loop(0, nchunks_w, chunk_body, 0)


def _sc_decode(zn, src, dst):
    mesh = plsc.VectorSubcoreMesh(core_axis_name="c", subcore_axis_name="s")
    return pl.kernel(
        _sc_body,
        out_type=jax.ShapeDtypeStruct((E,), jnp.float32),
        mesh=mesh,
        scratch_types=[
            pltpu.VMEM((C,), jnp.int32),
            pltpu.VMEM((C,), jnp.int32),
            pltpu.VMEM((C, D), jnp.float32),
            pltpu.VMEM((C, D), jnp.float32),
            pltpu.VMEM((C * L,), jnp.float32),
            pltpu.VMEM((C,), jnp.float32),
            pltpu.SemaphoreType.DMA,
        ],
        compiler_params=pltpu.CompilerParams(needs_layout_passes=False),
    )(zn, src, dst)


def kernel(z, edge_index):
    src = edge_index[0].astype(jnp.int32)
    dst = edge_index[1].astype(jnp.int32)
    zn = _normalize(z)
    return _sc_decode(zn, src, dst)


# speedup vs baseline: 3.3627x; 1.2412x over previous
import jax
import jax.numpy as jnp
from jax import lax
from jax.experimental import pallas as pl
from jax.experimental.pallas import tpu as pltpu
from jax.experimental.pallas import tpu_sc as plsc

N = 10000
D = 256
E = 160000
L = 16
NC = 2
NS = 16
NW = NC * NS
C = 128
NCHUNKS = E // C
NCW = (NCHUNKS + NW - 1) // NW
JB = D // 32


def _normalize_body(z_ref, out_ref):
    z = z_ref[...]
    ss = jnp.sum(z * z, axis=1, keepdims=True)
    inv = 1.0 / jnp.maximum(jnp.sqrt(ss), 1e-8)
    out_ref[...] = (z * inv).astype(jnp.bfloat16)


def _normalize(z):
    return pl.pallas_call(
        _normalize_body,
        out_shape=jax.ShapeDtypeStruct((N, D), jnp.bfloat16),
    )(z)


def _sc_body(zn_hbm, idx_hbm, out_hbm,
             i0, i1, a0, a1, b0, b1, p_v, out_v, sem0, sem1):
    wid = lax.axis_index("s") * NC + lax.axis_index("c")
    bufs = ((i0, a0, b0, sem0), (i1, a1, b1, sem1))

    def chunk_base(k):
        ci = jnp.minimum(wid + k * NW, NCHUNKS - 1)
        return ci * C

    def issue(k, b):
        iv, av, bv, sem = bufs[b]
        cb = chunk_base(k)
        pltpu.sync_copy(idx_hbm.at[pl.ds(cb * 2, 2 * C)], iv)
        pltpu.async_copy(zn_hbm.at[iv.at[pl.ds(0, C)]], av, sem)
        pltpu.async_copy(zn_hbm.at[iv.at[pl.ds(C, C)]], bv, sem)

    def wait(b):
        iv, av, bv, sem = bufs[b]
        pltpu.make_async_copy(zn_hbm.at[iv.at[pl.ds(0, C)]], av, sem).wait()
        pltpu.make_async_copy(zn_hbm.at[iv.at[pl.ds(C, C)]], bv, sem).wait()

    lanes = lax.iota(jnp.int32, L)

    def compute(k, b):
        iv, av, bv, sem = bufs[b]
        cb = chunk_base(k)

        def edge_body(e, _):
            acc = None
            for j in range(JB):
                va = plsc.bitcast(av[e, pl.ds(j * L, L)], jnp.bfloat16)
                vb = plsc.bitcast(bv[e, pl.ds(j * L, L)], jnp.bfloat16)
                a_lo, a_hi = plsc.unpack(va, format=plsc.PackFormat.INTERLEAVED)
                b_lo, b_hi = plsc.unpack(vb, format=plsc.PackFormat.INTERLEAVED)
                prod = a_lo * b_lo + a_hi * b_hi
                acc = prod if acc is None else acc + prod
            p_v[pl.ds(e * L, L)] = acc
            return 0

        lax.fori_loop(0, C, edge_body, 0)

        for g in range(C // L):
            base_idx = (lanes + g * L) * L
            acc = plsc.load_gather(p_v, [base_idx])
            for d in range(1, L):
                acc = acc + plsc.load_gather(p_v, [base_idx + d])
            sig = 1.0 / (1.0 + jnp.exp(-acc))
            out_v[pl.ds(g * L, L)] = sig

        pltpu.sync_copy(out_v, out_hbm.at[pl.ds(cb, C)])

    issue(0, 0)

    def step(i, _):
        k0 = 2 * i
        k1 = 2 * i + 1
        wait(0)
        issue(k1, 1)
        compute(k0, 0)
        wait(1)

        @pl.when(k1 < NCW - 1)
        def _():
            issue(k1 + 1, 0)

        compute(k1, 1)
        return 0

    lax.fori_loop(0, NCW // 2, step, 0)


def _sc_decode(zn, idx):
    mesh = plsc.VectorSubcoreMesh(core_axis_name="c", subcore_axis_name="s")
    return pl.kernel(
        _sc_body,
        out_type=jax.ShapeDtypeStruct((E,), jnp.float32),
        mesh=mesh,
        scratch_types=[
            pltpu.VMEM((2 * C,), jnp.int32),
            pltpu.VMEM((2 * C,), jnp.int32),
            pltpu.VMEM((C, D // 2), jnp.int32),
            pltpu.VMEM((C, D // 2), jnp.int32),
            pltpu.VMEM((C, D // 2), jnp.int32),
            pltpu.VMEM((C, D // 2), jnp.int32),
            pltpu.VMEM((C * L,), jnp.float32),
            pltpu.VMEM((C,), jnp.float32),
            pltpu.SemaphoreType.DMA,
            pltpu.SemaphoreType.DMA,
        ],
        compiler_params=pltpu.CompilerParams(needs_layout_passes=False),
    )(zn, idx)


def kernel(z, edge_index):
    idx = edge_index.astype(jnp.int32).reshape(2, NCHUNKS, C)
    idx = jnp.transpose(idx, (1, 0, 2)).reshape(2 * E)
    zn = _normalize(z)
    zn_i32 = jax.lax.bitcast_convert_type(zn.reshape(N, D // 2, 2), jnp.int32)
    return _sc_decode(zn_i32, idx)
